# baseline (speedup 1.0000x reference)
import functools

import jax
import jax.numpy as jnp
from jax import lax
from jax.experimental import pallas as pl
from jax.experimental.pallas import tpu as pltpu
from jax.experimental.pallas import tpu_sc as plsc

N, E, D, H, G = 10000, 320000, 128, 128, 64
NC, NS = 2, 16
NW = NC * NS
EPW = E // NW
CH = 96
NCHUNK = 105
EPWP = NCHUNK * CH
NPAD = 10240
RPW = NPAD // NS
BN = 2000
NBLK = N // BN


def _segsum_sc(x, srcf, dste, zeros):
    mesh = plsc.VectorSubcoreMesh(core_axis_name="c", subcore_axis_name="s")

    @functools.partial(
        pl.kernel,
        out_type=jax.ShapeDtypeStruct((NC, NPAD, D), jnp.float32),
        mesh=mesh,
        scratch_types=[
            pltpu.VMEM((EPWP,), jnp.int32),
            pltpu.VMEM((NCHUNK, CH), jnp.int32),
            pltpu.VMEM((CH, D), jnp.float32),
            pltpu.VMEM((CH, D), jnp.float32),
            pltpu.VMEM_SHARED((NPAD, D), jnp.float32),
            pltpu.SemaphoreType.DMA,
            pltpu.SemaphoreType.DMA,
        ],
    )
    def k(x_hbm, src_hbm, dst_hbm, z_hbm, out_hbm, src_v, dst_v, rows0_v, rows1_v,
          aggr_s, sem0, sem1):
        cid = lax.axis_index("c")
        sid = lax.axis_index("s")
        w = cid * NS + sid
        pltpu.sync_copy(src_hbm.at[w], src_v)
        pltpu.sync_copy(dst_hbm.at[w], dst_v)
        pltpu.sync_copy(z_hbm.at[pl.ds(sid * RPW, RPW)],
                        aggr_s.at[pl.ds(sid * RPW, RPW)])
        plsc.subcore_barrier()

        def start(j, rows_v, sem):
            pltpu.make_async_copy(
                x_hbm.at[src_v.at[pl.ds(j * CH, CH)]], rows_v, sem).start()

        def wait(j, rows_v, sem):
            pltpu.make_async_copy(
                x_hbm.at[src_v.at[pl.ds(j * CH, CH)]], rows_v, sem).wait()

        def scat(j, rows_v):
            pltpu.sync_copy(rows_v, aggr_s.at[dst_v.at[j]], add=True)

        start(0, rows0_v, sem0)

        def body(t, carry):
            j = 2 * t
            wait(j, rows0_v, sem0)

            @pl.when(j + 1 < NCHUNK)
            def _():
                start(j + 1, rows1_v, sem1)

            scat(j, rows0_v)

            @pl.when(j + 1 < NCHUNK)
            def _():
                wait(j + 1, rows1_v, sem1)

                @pl.when(j + 2 < NCHUNK)
                def _():
                    start(j + 2, rows0_v, sem0)

                scat(j + 1, rows1_v)

            return carry

        lax.fori_loop(0, (NCHUNK + 1) // 2, body, 0)
        plsc.subcore_barrier()
        pltpu.sync_copy(aggr_s.at[pl.ds(sid * RPW, RPW)],
                        out_hbm.at[cid, pl.ds(sid * RPW, RPW)])

    return k(x, srcf, dste, zeros)


def _dotT(a, w):
    return lax.dot_general(a, w, (((1,), (1,)), ((), ())),
                           preferred_element_type=jnp.float32)


def _layer_tc(p, x, w_rel, b_rel, w_root):
    def body(p0_r, p1_r, x_r, wr_r, br_r, wo_r, o_r):
        a = p0_r[0] + p1_r[0]
        h = _dotT(a, wr_r[...]) + _dotT(x_r[...], wo_r[...]) + br_r[...]
        o_r[...] = jnp.maximum(h, 0.0)

    full = lambda i: (0, 0)
    return pl.pallas_call(
        body,
        grid=(NBLK,),
        in_specs=[
            pl.BlockSpec((1, BN, D), lambda i: (0, i, 0)),
            pl.BlockSpec((1, BN, D), lambda i: (1, i, 0)),
            pl.BlockSpec((BN, D), lambda i: (i, 0)),
            pl.BlockSpec((H, D), full),
            pl.BlockSpec((1, H), full),
            pl.BlockSpec((H, D), full),
        ],
        out_specs=pl.BlockSpec((BN, H), lambda i: (i, 0)),
        out_shape=jax.ShapeDtypeStruct((N, H), jnp.float32),
    )(p, p, x, w_rel, b_rel, w_root)


def _final_tc(p, h1, w_rel, b_rel, w_root, batch3d,
              w1m, w1a, b1, w2t, b2):
    def body(p0_r, p1_r, h_r, wr_r, br_r, wo_r, bt_r,
             w1m_r, w1a_r, b1_r, w2_r, b2_r, o_r, accp, accc):
        i = pl.program_id(0)

        @pl.when(i == 0)
        def _():
            accp[...] = jnp.zeros_like(accp)
            accc[...] = jnp.zeros_like(accc)

        a = p0_r[0] + p1_r[0]
        h2 = jnp.maximum(_dotT(a, wr_r[...]) + _dotT(h_r[...], wo_r[...])
                         + br_r[...], 0.0)
        b = bt_r[...].reshape(1, BN)
        gi = lax.broadcasted_iota(jnp.int32, (G, BN), 0)
        maskf = (gi == b).astype(jnp.float32)
        accp[...] += lax.dot_general(maskf, h2, (((1,), (0,)), ((), ())),
                                     preferred_element_type=jnp.float32)
        accc[...] = accc[...] + jnp.sum(maskf, axis=1, keepdims=True)

        @pl.when(i == NBLK - 1)
        def _():
            addp = accp[...]
            meanp = addp / jnp.maximum(accc[...], 1.0)
            r = jnp.maximum(_dotT(meanp, w1m_r[...]) + _dotT(addp, w1a_r[...])
                            + b1_r[...], 0.0)
            o_r[...] = lax.dot_general(r, w2_r[...], (((1,), (0,)), ((), ())),
                                       preferred_element_type=jnp.float32) \
                       + b2_r[...]

    full = lambda i: (0, 0)
    return pl.pallas_call(
        body,
        grid=(NBLK,),
        in_specs=[
            pl.BlockSpec((1, BN, D), lambda i: (0, i, 0)),
            pl.BlockSpec((1, BN, D), lambda i: (1, i, 0)),
            pl.BlockSpec((BN, D), lambda i: (i, 0)),
            pl.BlockSpec((H, D), full),
            pl.BlockSpec((1, H), full),
            pl.BlockSpec((H, D), full),
            pl.BlockSpec((1, 1, BN), lambda i: (i, 0, 0)),
            pl.BlockSpec((H, H), full),
            pl.BlockSpec((H, H), full),
            pl.BlockSpec((1, H), full),
            pl.BlockSpec((H, 1), full),
            pl.BlockSpec((1, 1), full),
        ],
        out_specs=pl.BlockSpec((G, 1), full),
        out_shape=jax.ShapeDtypeStruct((G, 1), jnp.float32),
        scratch_shapes=[
            pltpu.VMEM((G, H), jnp.float32),
            pltpu.VMEM((G, H), jnp.float32),
        ],
    )(p, p, h1, w_rel, b_rel, w_root, batch3d, w1m, w1a, b1, w2t, b2)


def kernel(x, edge_index, batch, W_rel1, b_rel1, W_root1,
           W_rel2, b_rel2, W_root2, Wh1, bh1, Wh2, bh2):
    pad = EPWP - EPW
    srcf = jnp.pad(edge_index[0].reshape(NW, EPW), ((0, 0), (0, pad)))
    padrow = (N + jnp.arange(NW, dtype=jnp.int32))[:, None]
    dste = jnp.concatenate(
        [edge_index[1].reshape(NW, EPW),
         jnp.broadcast_to(padrow, (NW, pad))], axis=1).reshape(NW, NCHUNK, CH)
    zeros = jnp.zeros((NPAD, D), jnp.float32)
    batch3d = batch.reshape(NBLK, 1, BN)

    p = _segsum_sc(x, srcf, dste, zeros)
    h1 = _layer_tc(p, x, W_rel1, b_rel1.reshape(1, H), W_root1)
    q = _segsum_sc(h1, srcf, dste, zeros)
    out = _final_tc(q, h1, W_rel2, b_rel2.reshape(1, H), W_root2,
                    batch3d, Wh1[:, :H], Wh1[:, H:], bh1.reshape(1, H),
                    Wh2.reshape(H, 1), bh2.reshape(1, 1))
    return out.reshape(G)


# speedup vs baseline: 1.4338x; 1.4338x over previous
import functools

import jax
import jax.numpy as jnp
from jax import lax
from jax.experimental import pallas as pl
from jax.experimental.pallas import tpu as pltpu
from jax.experimental.pallas import tpu_sc as plsc

N, E, D, H, G = 10000, 320000, 128, 128, 64
NC, NS = 2, 16
NW = NC * NS
EPW = E // NW
CH = 80
NCHUNK = EPW // CH
EPWP = EPW
NPAD = 10240
RPW = NPAD // NS
BN = 2000
NBLK = N // BN


def _segsum_sc(x, srcf, dste, zeros):
    mesh = plsc.VectorSubcoreMesh(core_axis_name="c", subcore_axis_name="s")

    @functools.partial(
        pl.kernel,
        out_type=jax.ShapeDtypeStruct((NC, NPAD, D), jnp.float32),
        mesh=mesh,
        scratch_types=[
            pltpu.VMEM((EPWP,), jnp.int32),
            pltpu.VMEM((NCHUNK, CH), jnp.int32),
            pltpu.VMEM((CH, D), jnp.float32),
            pltpu.VMEM((CH, D), jnp.float32),
            pltpu.VMEM_SHARED((NPAD, D), jnp.float32),
            pltpu.SemaphoreType.DMA,
            pltpu.SemaphoreType.DMA,
        ],
    )
    def k(x_hbm, src_hbm, dst_hbm, z_hbm, out_hbm, src_v, dst_v, rows0_v, rows1_v,
          aggr_s, sem0, sem1):
        cid = lax.axis_index("c")
        sid = lax.axis_index("s")
        w = cid * NS + sid
        pltpu.sync_copy(src_hbm.at[w], src_v)
        pltpu.sync_copy(dst_hbm.at[w], dst_v)
        pltpu.sync_copy(z_hbm.at[pl.ds(sid * RPW, RPW)],
                        aggr_s.at[pl.ds(sid * RPW, RPW)])
        plsc.subcore_barrier()

        HC = CH // 2

        def start(j, rows_v, sem):
            pltpu.make_async_copy(
                x_hbm.at[src_v.at[pl.ds(j * CH, HC)]],
                rows_v.at[pl.ds(0, HC)], sem).start()
            pltpu.make_async_copy(
                x_hbm.at[src_v.at[pl.ds(j * CH + HC, HC)]],
                rows_v.at[pl.ds(HC, HC)], sem).start()

        def wait(j, rows_v, sem):
            cp = pltpu.make_async_copy(
                x_hbm.at[src_v.at[pl.ds(j * CH, HC)]],
                rows_v.at[pl.ds(0, HC)], sem)
            cp.wait()
            cp.wait()

        def scat(j, rows_v):
            pltpu.sync_copy(rows_v, aggr_s.at[dst_v.at[j]], add=True)

        start(0, rows0_v, sem0)

        def body(t, carry):
            j = 2 * t
            wait(j, rows0_v, sem0)

            @pl.when(j + 1 < NCHUNK)
            def _():
                start(j + 1, rows1_v, sem1)

            scat(j, rows0_v)

            @pl.when(j + 1 < NCHUNK)
            def _():
                wait(j + 1, rows1_v, sem1)

                @pl.when(j + 2 < NCHUNK)
                def _():
                    start(j + 2, rows0_v, sem0)

                scat(j + 1, rows1_v)

            return carry

        lax.fori_loop(0, (NCHUNK + 1) // 2, body, 0)
        plsc.subcore_barrier()
        pltpu.sync_copy(aggr_s.at[pl.ds(sid * RPW, RPW)],
                        out_hbm.at[cid, pl.ds(sid * RPW, RPW)])

    return k(x, srcf, dste, zeros)


def _dotT(a, w):
    return lax.dot_general(a, w, (((1,), (1,)), ((), ())),
                           preferred_element_type=jnp.float32)


def _layer_tc(p, x, w_rel, b_rel, w_root):
    def body(p0_r, p1_r, x_r, wr_r, br_r, wo_r, o_r):
        a = p0_r[0] + p1_r[0]
        h = _dotT(a, wr_r[...]) + _dotT(x_r[...], wo_r[...]) + br_r[...]
        o_r[...] = jnp.maximum(h, 0.0)

    full = lambda i: (0, 0)
    return pl.pallas_call(
        body,
        grid=(NBLK,),
        in_specs=[
            pl.BlockSpec((1, BN, D), lambda i: (0, i, 0)),
            pl.BlockSpec((1, BN, D), lambda i: (1, i, 0)),
            pl.BlockSpec((BN, D), lambda i: (i, 0)),
            pl.BlockSpec((H, D), full),
            pl.BlockSpec((1, H), full),
            pl.BlockSpec((H, D), full),
        ],
        out_specs=pl.BlockSpec((BN, H), lambda i: (i, 0)),
        out_shape=jax.ShapeDtypeStruct((N, H), jnp.float32),
    )(p, p, x, w_rel, b_rel, w_root)


def _final_tc(p, h1, w_rel, b_rel, w_root, batch3d,
              w1m, w1a, b1, w2t, b2):
    def body(p0_r, p1_r, h_r, wr_r, br_r, wo_r, bt_r,
             w1m_r, w1a_r, b1_r, w2_r, b2_r, o_r, accp, accc):
        i = pl.program_id(0)

        @pl.when(i == 0)
        def _():
            accp[...] = jnp.zeros_like(accp)
            accc[...] = jnp.zeros_like(accc)

        a = p0_r[0] + p1_r[0]
        h2 = jnp.maximum(_dotT(a, wr_r[...]) + _dotT(h_r[...], wo_r[...])
                         + br_r[...], 0.0)
        b = bt_r[...].reshape(1, BN)
        gi = lax.broadcasted_iota(jnp.int32, (G, BN), 0)
        maskf = (gi == b).astype(jnp.float32)
        accp[...] += lax.dot_general(maskf, h2, (((1,), (0,)), ((), ())),
                                     preferred_element_type=jnp.float32)
        accc[...] = accc[...] + jnp.sum(maskf, axis=1, keepdims=True)

        @pl.when(i == NBLK - 1)
        def _():
            addp = accp[...]
            meanp = addp / jnp.maximum(accc[...], 1.0)
            r = jnp.maximum(_dotT(meanp, w1m_r[...]) + _dotT(addp, w1a_r[...])
                            + b1_r[...], 0.0)
            o_r[...] = lax.dot_general(r, w2_r[...], (((1,), (0,)), ((), ())),
                                       preferred_element_type=jnp.float32) \
                       + b2_r[...]

    full = lambda i: (0, 0)
    return pl.pallas_call(
        body,
        grid=(NBLK,),
        in_specs=[
            pl.BlockSpec((1, BN, D), lambda i: (0, i, 0)),
            pl.BlockSpec((1, BN, D), lambda i: (1, i, 0)),
            pl.BlockSpec((BN, D), lambda i: (i, 0)),
            pl.BlockSpec((H, D), full),
            pl.BlockSpec((1, H), full),
            pl.BlockSpec((H, D), full),
            pl.BlockSpec((1, 1, BN), lambda i: (i, 0, 0)),
            pl.BlockSpec((H, H), full),
            pl.BlockSpec((H, H), full),
            pl.BlockSpec((1, H), full),
            pl.BlockSpec((H, 1), full),
            pl.BlockSpec((1, 1), full),
        ],
        out_specs=pl.BlockSpec((G, 1), full),
        out_shape=jax.ShapeDtypeStruct((G, 1), jnp.float32),
        scratch_shapes=[
            pltpu.VMEM((G, H), jnp.float32),
            pltpu.VMEM((G, H), jnp.float32),
        ],
    )(p, p, h1, w_rel, b_rel, w_root, batch3d, w1m, w1a, b1, w2t, b2)


def kernel(x, edge_index, batch, W_rel1, b_rel1, W_root1,
           W_rel2, b_rel2, W_root2, Wh1, bh1, Wh2, bh2):
    srcf = edge_index[0].reshape(NW, EPW)
    dste = edge_index[1].reshape(NW, NCHUNK, CH)
    zeros = jnp.zeros((NPAD, D), jnp.float32)
    batch3d = batch.reshape(NBLK, 1, BN)

    p = _segsum_sc(x, srcf, dste, zeros)
    h1 = _layer_tc(p, x, W_rel1, b_rel1.reshape(1, H), W_root1)
    q = _segsum_sc(h1, srcf, dste, zeros)
    out = _final_tc(q, h1, W_rel2, b_rel2.reshape(1, H), W_root2,
                    batch3d, Wh1[:, :H], Wh1[:, H:], bh1.reshape(1, H),
                    Wh2.reshape(H, 1), bh2.reshape(1, 1))
    return out.reshape(G)
